# Initial kernel scaffold; baseline (speedup 1.0000x reference)
import jax
import jax.numpy as jnp
from jax.experimental import pallas as pl


def kernel(x, tok_table, pos_table):
    raise NotImplementedError("write your pallas kernel here")


# speedup vs baseline: 1.5787x; 1.5787x over previous
import functools

import jax
import jax.numpy as jnp
from jax import lax
from jax.experimental import pallas as pl
from jax.experimental.pallas import tpu as pltpu
from jax.experimental.pallas import tpu_sc as plsc

B, L, D = 4096, 200, 64
NUM_CORES, NUM_SUBCORES = 2, 16
NUM_WORKERS = NUM_CORES * NUM_SUBCORES
ROWS_PER_WORKER = B // NUM_WORKERS
LANES = 16
SEG1, SEG2, SEG2_PAD = 128, 72, 80


def _emb_body(x_ref, tok_ref, pos_ref, out_ref,
              xb1, xb2, pb1, pb2, tb1, tb2, qb1, qb2, sem):
    wid = lax.axis_index("s") * NUM_CORES + lax.axis_index("c")
    lane = lax.iota(jnp.int32, LANES)

    def row_body(r, _):
        gr = wid * ROWS_PER_WORKER + r
        base = gr * L
        pltpu.sync_copy(x_ref.at[pl.ds(base, SEG1)], xb1)
        pltpu.sync_copy(x_ref.at[pl.ds(base + SEG1, SEG2)],
                        xb2.at[pl.ds(0, SEG2)])

        carry = jnp.zeros((LANES,), jnp.int32)
        for c in range(SEG1 // LANES):
            v = xb1[pl.ds(c * LANES, LANES)]
            m = v != 0
            cs = plsc.cumsum(m.astype(jnp.int32))
            pb1[pl.ds(c * LANES, LANES)] = jnp.where(m, cs + carry, 0)
            carry = carry + plsc.all_reduce_population_count(m)
        for c in range(SEG2_PAD // LANES):
            v = xb2[pl.ds(c * LANES, LANES)]
            if c == SEG2_PAD // LANES - 1:
                v = jnp.where(lane < SEG2 - c * LANES, v, 0)
                xb2[pl.ds(c * LANES, LANES)] = v
            m = v != 0
            cs = plsc.cumsum(m.astype(jnp.int32))
            pb2[pl.ds(c * LANES, LANES)] = jnp.where(m, cs + carry, 0)
            carry = carry + plsc.all_reduce_population_count(m)

        cps = [
            pltpu.async_copy(tok_ref.at[xb1], tb1, sem),
            pltpu.async_copy(tok_ref.at[xb2], tb2, sem),
            pltpu.async_copy(pos_ref.at[pb1], qb1, sem),
            pltpu.async_copy(pos_ref.at[pb2], qb2, sem),
        ]
        for cp in cps:
            cp.wait()

        def add1(l, c):
            for j in range(D // LANES):
                s = pl.ds(j * LANES, LANES)
                tb1[l, s] = tb1[l, s] + qb1[l, s]
            return c
        lax.fori_loop(0, SEG1, add1, 0)

        def add2(l, c):
            for j in range(D // LANES):
                s = pl.ds(j * LANES, LANES)
                tb2[l, s] = tb2[l, s] + qb2[l, s]
            return c
        lax.fori_loop(0, SEG2, add2, 0)

        pltpu.sync_copy(tb1, out_ref.at[gr].at[pl.ds(0, SEG1)])
        pltpu.sync_copy(tb2.at[pl.ds(0, SEG2)],
                        out_ref.at[gr].at[pl.ds(SEG1, SEG2)])
        return _

    lax.fori_loop(0, ROWS_PER_WORKER, row_body, 0)


@jax.jit
def _emb(x, tok_table, pos_table):
    mesh = plsc.VectorSubcoreMesh(core_axis_name="c", subcore_axis_name="s")
    f = functools.partial(
        pl.kernel,
        out_type=jax.ShapeDtypeStruct((B, L, D), jnp.float32),
        mesh=mesh,
        scratch_types=[
            pltpu.VMEM((SEG1,), jnp.int32),
            pltpu.VMEM((SEG2_PAD,), jnp.int32),
            pltpu.VMEM((SEG1,), jnp.int32),
            pltpu.VMEM((SEG2_PAD,), jnp.int32),
            pltpu.VMEM((SEG1, D), jnp.float32),
            pltpu.VMEM((SEG2_PAD, D), jnp.float32),
            pltpu.VMEM((SEG1, D), jnp.float32),
            pltpu.VMEM((SEG2_PAD, D), jnp.float32),
            pltpu.SemaphoreType.DMA,
        ],
        compiler_params=pltpu.CompilerParams(
            needs_layout_passes=False, use_tc_tiling_on_sc=False),
    )(_emb_body)
    return f(x.reshape(-1), tok_table, pos_table)


def kernel(x, tok_table, pos_table):
    x = x.astype(jnp.int32)
    out = _emb(x, tok_table, pos_table)
    return out, x == 0
